# Initial kernel scaffold; baseline (speedup 1.0000x reference)
import jax
import jax.numpy as jnp
from jax.experimental import pallas as pl


def kernel(x_embed, prompt, prompt_key):
    raise NotImplementedError("write your pallas kernel here")


# speedup vs baseline: 2.3237x; 2.3237x over previous
import functools

import jax
import jax.numpy as jnp
from jax import lax
from jax.experimental import pallas as pl
from jax.experimental.pallas import tpu as pltpu
from jax.experimental.pallas import tpu_sc as plsc

POOL = 8192
PLEN = 5
D = 768
TOPK = 8
BATCH = 1024
SEQ = 64
PROMPT_ROWS = TOPK * PLEN
TAIL = SEQ - PROMPT_ROWS

_NW = 32
_BPW = BATCH // _NW


def _mean_norm_body(x_ref, o_ref):
    x = x_ref[...]
    m = jnp.mean(x, axis=1)
    ss = jnp.sum(m * m, axis=1, keepdims=True)
    o_ref[...] = m * lax.rsqrt(jnp.maximum(ss, 1e-12))


def _mean_norm(x):
    bt = 64
    return pl.pallas_call(
        _mean_norm_body,
        grid=(BATCH // bt,),
        in_specs=[pl.BlockSpec((bt, SEQ, D), lambda i: (i, 0, 0))],
        out_specs=pl.BlockSpec((bt, D), lambda i: (i, 0)),
        out_shape=jax.ShapeDtypeStruct((BATCH, D), jnp.float32),
    )(x)


def _pk_norm_body(k_ref, o_ref):
    k = k_ref[...]
    ss = jnp.sum(k * k, axis=1, keepdims=True)
    o_ref[...] = k * lax.rsqrt(jnp.maximum(ss, 1e-12))


def _pk_norm(pk):
    rt = 1024
    return pl.pallas_call(
        _pk_norm_body,
        grid=(POOL // rt,),
        in_specs=[pl.BlockSpec((rt, D), lambda i: (i, 0))],
        out_specs=pl.BlockSpec((rt, D), lambda i: (i, 0)),
        out_shape=jax.ShapeDtypeStruct((POOL, D), jnp.float32),
    )(pk)


def _sim_body(x_ref, p_ref, o_ref):
    o_ref[...] = lax.dot_general(
        x_ref[...], p_ref[...],
        (((1,), (1,)), ((), ())),
        preferred_element_type=jnp.float32)


def _similarity(xn, pn):
    bt, pt = 256, 2048
    return pl.pallas_call(
        _sim_body,
        grid=(BATCH // bt, POOL // pt),
        in_specs=[pl.BlockSpec((bt, D), lambda i, j: (i, 0)),
                  pl.BlockSpec((pt, D), lambda i, j: (j, 0))],
        out_specs=pl.BlockSpec((bt, pt), lambda i, j: (i, j)),
        out_shape=jax.ShapeDtypeStruct((BATCH, POOL), jnp.float32),
    )(xn, pn)


def _topk_body(s_ref, idx_ref, psum_ref, scr_ref):
    bt = s_ref.shape[0]
    scr_ref[...] = s_ref[...]
    cols = lax.broadcasted_iota(jnp.int32, (bt, POOL), 1)
    ids = []
    tot = None
    for _ in range(TOPK):
        cur = scr_ref[...]
        m = jnp.max(cur, axis=1, keepdims=True)
        am = jnp.min(jnp.where(cur == m, cols, POOL), axis=1,
                     keepdims=True)
        ids.append(am)
        tot = m if tot is None else tot + m
        scr_ref[...] = jnp.where(cols == am, -jnp.inf, cur)
    idx_ref[...] = jnp.concatenate(ids, axis=1)
    psum_ref[...] = jnp.sum(tot)[None, None, None]


def _topk(sim):
    bt = 256
    nt = BATCH // bt
    return pl.pallas_call(
        _topk_body,
        grid=(nt,),
        in_specs=[pl.BlockSpec((bt, POOL), lambda i: (i, 0))],
        out_specs=[pl.BlockSpec((bt, TOPK), lambda i: (i, 0)),
                   pl.BlockSpec((1, 1, 1), lambda i: (i, 0, 0))],
        out_shape=[jax.ShapeDtypeStruct((BATCH, TOPK), jnp.int32),
                   jax.ShapeDtypeStruct((nt, 1, 1), jnp.float32)],
        scratch_shapes=[pltpu.VMEM((bt, POOL), jnp.float32)],
    )(sim)


def _sc_assemble(x, prompt2, pn, idx40, idx8):
    mesh = plsc.VectorSubcoreMesh(core_axis_name="c", subcore_axis_name="s")

    @functools.partial(
        pl.kernel,
        out_type=[jax.ShapeDtypeStruct((BATCH, SEQ, D), jnp.float32),
                  jax.ShapeDtypeStruct((BATCH, TOPK, D), jnp.float32)],
        mesh=mesh,
        scratch_types=[pltpu.VMEM((_BPW, PROMPT_ROWS), jnp.int32),
                       pltpu.VMEM((_BPW, TOPK), jnp.int32),
                       pltpu.VMEM((PROMPT_ROWS, D), jnp.float32),
                       pltpu.VMEM((TAIL, D), jnp.float32),
                       pltpu.VMEM((TOPK, D), jnp.float32),
                       pltpu.SemaphoreType.DMA,
                       pltpu.SemaphoreType.DMA,
                       pltpu.SemaphoreType.DMA],
    )
    def k(x_hbm, prompt_hbm, pn_hbm, i40_hbm, i8_hbm, out_hbm, bkn_hbm,
          i40_v, i8_v, pbuf, tbuf, kbuf, sem0, sem1, sem2):
        wid = lax.axis_index("c") * 16 + lax.axis_index("s")
        base = wid * _BPW
        pltpu.sync_copy(i40_hbm.at[pl.ds(base, _BPW)], i40_v)
        pltpu.sync_copy(i8_hbm.at[pl.ds(base, _BPW)], i8_v)

        @pl.loop(0, _BPW)
        def _(bl):
            b = base + bl
            c1 = pltpu.async_copy(prompt_hbm.at[i40_v.at[bl]], pbuf, sem0)
            c2 = pltpu.async_copy(x_hbm.at[b, pl.ds(PROMPT_ROWS, TAIL)],
                                  tbuf, sem1)
            c3 = pltpu.async_copy(pn_hbm.at[i8_v.at[bl]], kbuf, sem2)
            c1.wait()
            c2.wait()
            c3.wait()
            pltpu.sync_copy(pbuf, out_hbm.at[b, pl.ds(0, PROMPT_ROWS)])
            pltpu.sync_copy(tbuf, out_hbm.at[b, pl.ds(PROMPT_ROWS, TAIL)])
            pltpu.sync_copy(kbuf, bkn_hbm.at[b])

    return k(x, prompt2, pn, idx40, idx8)


def kernel(x_embed, prompt, prompt_key):
    xn = _mean_norm(x_embed)
    pn = _pk_norm(prompt_key)
    sim = _similarity(xn, pn)
    idx, psums = _topk(sim)
    reduce_sim = jnp.sum(psums) / BATCH
    idx40 = (idx[:, :, None] * PLEN
             + jnp.arange(PLEN, dtype=jnp.int32)).reshape(BATCH, PROMPT_ROWS)
    prompt2 = prompt.reshape(POOL * PLEN, D)
    out, bkn = _sc_assemble(x_embed, prompt2, pn, idx40, idx)
    return (out, reduce_sim, sim, idx, bkn)


# speedup vs baseline: 2.4413x; 1.0506x over previous
import functools

import jax
import jax.numpy as jnp
from jax import lax
from jax.experimental import pallas as pl
from jax.experimental.pallas import tpu as pltpu
from jax.experimental.pallas import tpu_sc as plsc

POOL = 8192
PLEN = 5
D = 768
TOPK = 8
BATCH = 1024
SEQ = 64
PROMPT_ROWS = TOPK * PLEN
TAIL = SEQ - PROMPT_ROWS

_NW = 32
_BPW = BATCH // _NW


def _mean_norm_body(x_ref, o_ref):
    x = x_ref[...]
    m = jnp.mean(x, axis=1)
    ss = jnp.sum(m * m, axis=1, keepdims=True)
    o_ref[...] = m * lax.rsqrt(jnp.maximum(ss, 1e-12))


def _mean_norm(x):
    bt = 64
    return pl.pallas_call(
        _mean_norm_body,
        grid=(BATCH // bt,),
        in_specs=[pl.BlockSpec((bt, SEQ, D), lambda i: (i, 0, 0))],
        out_specs=pl.BlockSpec((bt, D), lambda i: (i, 0)),
        out_shape=jax.ShapeDtypeStruct((BATCH, D), jnp.float32),
    )(x)


def _pk_norm_body(k_ref, o_ref):
    k = k_ref[...]
    ss = jnp.sum(k * k, axis=1, keepdims=True)
    o_ref[...] = k * lax.rsqrt(jnp.maximum(ss, 1e-12))


def _pk_norm(pk):
    rt = 1024
    return pl.pallas_call(
        _pk_norm_body,
        grid=(POOL // rt,),
        in_specs=[pl.BlockSpec((rt, D), lambda i: (i, 0))],
        out_specs=pl.BlockSpec((rt, D), lambda i: (i, 0)),
        out_shape=jax.ShapeDtypeStruct((POOL, D), jnp.float32),
    )(pk)


def _sim_body(x_ref, p_ref, o_ref):
    o_ref[...] = lax.dot_general(
        x_ref[...], p_ref[...],
        (((1,), (1,)), ((), ())),
        preferred_element_type=jnp.float32)


def _similarity(xn, pn):
    bt, pt = 256, 2048
    return pl.pallas_call(
        _sim_body,
        grid=(POOL // pt, BATCH // bt),
        in_specs=[pl.BlockSpec((bt, D), lambda j, i: (i, 0)),
                  pl.BlockSpec((pt, D), lambda j, i: (j, 0))],
        out_specs=pl.BlockSpec((bt, pt), lambda j, i: (i, j)),
        out_shape=jax.ShapeDtypeStruct((BATCH, POOL), jnp.float32),
    )(xn, pn)


def _topk_body(s_ref, idx_ref, psum_ref):
    bt = s_ref.shape[0]
    cols = lax.broadcasted_iota(jnp.int32, (bt, POOL), 1)
    cur = s_ref[...]
    ids = []
    tot = None
    for j in range(TOPK):
        m = jnp.max(cur, axis=1, keepdims=True)
        am = jnp.min(jnp.where(cur == m, cols, POOL), axis=1,
                     keepdims=True)
        ids.append(am)
        tot = m if tot is None else tot + m
        if j < TOPK - 1:
            cur = jnp.where(cols == am, -jnp.inf, cur)
    idx_ref[...] = jnp.concatenate(ids, axis=1)
    psum_ref[...] = jnp.sum(tot)[None, None, None]


def _topk(sim):
    bt = 256
    nt = BATCH // bt
    return pl.pallas_call(
        _topk_body,
        grid=(nt,),
        in_specs=[pl.BlockSpec((bt, POOL), lambda i: (i, 0))],
        out_specs=[pl.BlockSpec((bt, TOPK), lambda i: (i, 0)),
                   pl.BlockSpec((1, 1, 1), lambda i: (i, 0, 0))],
        out_shape=[jax.ShapeDtypeStruct((BATCH, TOPK), jnp.int32),
                   jax.ShapeDtypeStruct((nt, 1, 1), jnp.float32)],
    )(sim)


def _sc_assemble(x, prompt2, pn, idx40, idx8):
    mesh = plsc.VectorSubcoreMesh(core_axis_name="c", subcore_axis_name="s")

    @functools.partial(
        pl.kernel,
        out_type=[jax.ShapeDtypeStruct((BATCH, SEQ, D), jnp.float32),
                  jax.ShapeDtypeStruct((BATCH, TOPK, D), jnp.float32)],
        mesh=mesh,
        scratch_types=[pltpu.VMEM((_BPW, PROMPT_ROWS), jnp.int32),
                       pltpu.VMEM((_BPW, TOPK), jnp.int32),
                       pltpu.VMEM((PROMPT_ROWS, D), jnp.float32),
                       pltpu.VMEM((PROMPT_ROWS, D), jnp.float32),
                       pltpu.VMEM((TAIL, D), jnp.float32),
                       pltpu.VMEM((TAIL, D), jnp.float32),
                       pltpu.VMEM((TOPK, D), jnp.float32),
                       pltpu.VMEM((TOPK, D), jnp.float32),
                       pltpu.SemaphoreType.DMA,
                       pltpu.SemaphoreType.DMA],
    )
    def k(x_hbm, prompt_hbm, pn_hbm, i40_hbm, i8_hbm, out_hbm, bkn_hbm,
          i40_v, i8_v, pbuf_a, pbuf_b, tbuf_a, tbuf_b, kbuf_a, kbuf_b,
          gsem_a, gsem_b):
        wid = lax.axis_index("c") * 16 + lax.axis_index("s")
        base = wid * _BPW
        pltpu.sync_copy(i40_hbm.at[pl.ds(base, _BPW)], i40_v)
        pltpu.sync_copy(i8_hbm.at[pl.ds(base, _BPW)], i8_v)

        def start(bl, pbuf, tbuf, kbuf, gsem):
            b = base + bl
            pltpu.async_copy(prompt_hbm.at[i40_v.at[bl]], pbuf, gsem)
            pltpu.async_copy(x_hbm.at[b, pl.ds(PROMPT_ROWS, TAIL)],
                             tbuf, gsem)
            pltpu.async_copy(pn_hbm.at[i8_v.at[bl]], kbuf, gsem)

        def finish(bl, pbuf, tbuf, kbuf, gsem):
            b = base + bl
            pltpu.make_async_copy(
                prompt_hbm.at[pl.ds(0, PROMPT_ROWS)], pbuf, gsem).wait()
            pltpu.make_async_copy(
                x_hbm.at[0, pl.ds(PROMPT_ROWS, TAIL)], tbuf, gsem).wait()
            pltpu.make_async_copy(
                pn_hbm.at[pl.ds(0, TOPK)], kbuf, gsem).wait()
            pltpu.sync_copy(pbuf, out_hbm.at[b, pl.ds(0, PROMPT_ROWS)])
            pltpu.sync_copy(tbuf, out_hbm.at[b, pl.ds(PROMPT_ROWS, TAIL)])
            pltpu.sync_copy(kbuf, bkn_hbm.at[b])

        start(0, pbuf_a, tbuf_a, kbuf_a, gsem_a)

        @pl.loop(0, _BPW, step=2)
        def _(bl):
            start(bl + 1, pbuf_b, tbuf_b, kbuf_b, gsem_b)
            finish(bl, pbuf_a, tbuf_a, kbuf_a, gsem_a)

            @pl.when(bl + 2 < _BPW)
            def _():
                start(bl + 2, pbuf_a, tbuf_a, kbuf_a, gsem_a)

            finish(bl + 1, pbuf_b, tbuf_b, kbuf_b, gsem_b)

    return k(x, prompt2, pn, idx40, idx8)


def kernel(x_embed, prompt, prompt_key):
    xn = _mean_norm(x_embed)
    pn = _pk_norm(prompt_key)
    sim = _similarity(xn, pn)
    idx, psums = _topk(sim)
    reduce_sim = jnp.sum(psums) / BATCH
    idx40 = (idx[:, :, None] * PLEN
             + jnp.arange(PLEN, dtype=jnp.int32)).reshape(BATCH, PROMPT_ROWS)
    prompt2 = prompt.reshape(POOL * PLEN, D)
    out, bkn = _sc_assemble(x_embed, prompt2, pn, idx40, idx)
    return (out, reduce_sim, sim, idx, bkn)
